# baseline (speedup 1.0000x reference)
import dataclasses
import functools

import jax
import jax.numpy as jnp
from jax import lax
from jax.experimental import pallas as pl
from jax.experimental.pallas import tpu as pltpu
from jax.experimental.pallas import tpu_sc as plsc

ROWS = 128
VOCAB = 100000
LANES = 16
NWORK = 32
CHC = 200
V_SC = 36000
NCHUNK = V_SC // CHC
NJOB = -(-NCHUNK // NWORK)
NBUF = 4
TCB = 4000
NSTREAM = 4
TC_STEPS = (VOCAB - V_SC) // (TCB * NSTREAM)
_INT_MAX = 2**31 - 1


def _sc_scan(xt):
    mesh = plsc.VectorSubcoreMesh(core_axis_name="c", subcore_axis_name="s")

    scratch = [pltpu.VMEM((CHC, ROWS), jnp.float32) for _ in range(NBUF)]
    scratch.append(pltpu.VMEM((ROWS,), jnp.float32))
    scratch.append(pltpu.VMEM((ROWS,), jnp.int32))
    scratch.extend(pltpu.SemaphoreType.DMA for _ in range(NBUF))
    n_full = NCHUNK - (NJOB - 1) * NWORK

    cp = pltpu.CompilerParams()
    if "needs_layout_passes" in pltpu.CompilerParams.__dataclass_fields__:
        cp = dataclasses.replace(cp, needs_layout_passes=False)

    @functools.partial(
        pl.kernel,
        out_type=(
            jax.ShapeDtypeStruct((NWORK, ROWS), jnp.float32),
            jax.ShapeDtypeStruct((NWORK, ROWS), jnp.int32),
        ),
        mesh=mesh,
        scratch_types=scratch,
        compiler_params=cp,
    )
    def sc_argmax(x_hbm, val_hbm, idx_hbm, *rest):
        bufs = rest[:NBUF]
        outv_f = rest[NBUF]
        outv_i = rest[NBUF + 1]
        sems = rest[NBUF + 2:]

        w = lax.axis_index("c") * 16 + lax.axis_index("s")
        has_last = w < n_full

        def _descr(j):
            off = pl.multiple_of((w + j * NWORK) * CHC, 8)
            return pltpu.make_async_copy(
                x_hbm.at[pl.ds(off, CHC), :],
                bufs[j % NBUF],
                sems[j % NBUF],
            )

        def issue(j):
            copy = _descr(j)
            copy.start()
            return copy

        def guarded_issue(j):
            if j < NJOB - 1:
                return issue(j)

            @pl.when(has_last)
            def _():
                issue(j)

        copies = {}
        for j in range(min(NBUF - 1, NJOB)):
            copies[j] = guarded_issue(j)

        neg_inf = jnp.full((LANES,), -jnp.inf, dtype=jnp.float32)
        zero_i = jnp.zeros((LANES,), dtype=jnp.int32)

        rms = [neg_inf] * 8
        ris = [zero_i] * 8

        for j in range(NJOB):
            nxt = j + (NBUF - 1)
            if nxt < NJOB:
                copies[nxt] = guarded_issue(nxt)
            last = j == NJOB - 1
            if not last:
                copies[j].wait()
            else:

                @pl.when(has_last)
                def _(j=j):
                    _descr(j).wait()

            buf = bufs[j % NBUF]
            cbase = (w + j * NWORK) * CHC
            madd = None
            if last:
                madd = jnp.where(
                    has_last, jnp.float32(0), jnp.float32(-jnp.inf)
                )

            def body(i, carry, buf=buf, cbase=cbase, madd=madd):
                c_rms, c_ris = carry
                c_rms, c_ris = list(c_rms), list(c_ris)
                col = jnp.broadcast_to(cbase + i, (LANES,)).astype(jnp.int32)
                for k in range(8):
                    v = buf[i, pl.ds(k * LANES, LANES)]
                    if madd is not None:
                        v = v + madd
                    m = v > c_rms[k]
                    c_rms[k] = jnp.where(m, v, c_rms[k])
                    c_ris[k] = jnp.where(m, col, c_ris[k])
                return tuple(c_rms), tuple(c_ris)

            rms_t, ris_t = lax.fori_loop(0, CHC, body, (tuple(rms), tuple(ris)))
            rms, ris = list(rms_t), list(ris_t)

        for k in range(8):
            outv_f[pl.ds(k * LANES, LANES)] = rms[k]
            outv_i[pl.ds(k * LANES, LANES)] = ris[k]
        pltpu.sync_copy(outv_f, val_hbm.at[w])
        pltpu.sync_copy(outv_i, idx_hbm.at[w])

    return sc_argmax(xt)


def _tc_scan(xt):
    def body(*refs):
        x_refs = refs[:NSTREAM]
        val_ref, idx_ref = refs[NSTREAM:NSTREAM + 2]
        accs = refs[NSTREAM + 2:]
        av = accs[:NSTREAM]
        ai = accs[NSTREAM:]
        pid = pl.program_id(0)

        @pl.when(pid == 0)
        def _():
            for a in range(NSTREAM):
                av[a][...] = jnp.full((8, ROWS), -jnp.inf, dtype=jnp.float32)
                ai[a][...] = jnp.zeros((8, ROWS), dtype=jnp.int32)

        sub_iota = lax.broadcasted_iota(jnp.int32, (8, ROWS), 0)
        cv = [av[a][...] for a in range(NSTREAM)]
        ci = [ai[a][...] for a in range(NSTREAM)]
        for s in range(NSTREAM):
            x = x_refs[s][...]
            base = V_SC + (pid * NSTREAM + s) * TCB
            for j in range(TCB // 8):
                xv = x[j * 8:(j + 1) * 8, :]
                iv = sub_iota + (base + j * 8)
                m = xv > cv[s]
                cv[s] = jnp.where(m, xv, cv[s])
                ci[s] = jnp.where(m, iv, ci[s])
        for a in range(NSTREAM):
            av[a][...] = cv[a]
            ai[a][...] = ci[a]

        @pl.when(pid == TC_STEPS - 1)
        def _():
            rv, ri = cv[0], ci[0]
            for a in range(1, NSTREAM):
                tb = (cv[a] > rv) | ((cv[a] == rv) & (ci[a] < ri))
                rv = jnp.where(tb, cv[a], rv)
                ri = jnp.where(tb, ci[a], ri)
            row_max = jnp.max(rv, axis=0, keepdims=True)
            cand = jnp.where(rv == row_max, ri, _INT_MAX)
            val_ref[...] = row_max
            idx_ref[...] = jnp.min(cand, axis=0, keepdims=True)

    in_specs = [
        pl.BlockSpec(
            (TCB, ROWS),
            functools.partial(
                lambda i, s: (V_SC // TCB + i * NSTREAM + s, 0), s=s
            ),
        )
        for s in range(NSTREAM)
    ]
    return pl.pallas_call(
        body,
        grid=(TC_STEPS,),
        out_shape=(
            jax.ShapeDtypeStruct((1, ROWS), jnp.float32),
            jax.ShapeDtypeStruct((1, ROWS), jnp.int32),
        ),
        in_specs=in_specs,
        out_specs=(
            pl.BlockSpec((1, ROWS), lambda i: (0, 0)),
            pl.BlockSpec((1, ROWS), lambda i: (0, 0)),
        ),
        scratch_shapes=[pltpu.VMEM((8, ROWS), jnp.float32)] * NSTREAM
        + [pltpu.VMEM((8, ROWS), jnp.int32)] * NSTREAM,
    )(*([xt] * NSTREAM))


def _merge(sc_val, sc_idx, tc_val, tc_idx):
    def body(sv_ref, si_ref, tv_ref, ti_ref, o_ref):
        sv = sv_ref[...]
        si = si_ref[...]
        smax = jnp.max(sv, axis=0, keepdims=True)
        scand = jnp.min(
            jnp.where(sv == smax, si, _INT_MAX), axis=0, keepdims=True
        )
        tv = tv_ref[...]
        ti = ti_ref[...]
        tb = (tv > smax) | ((tv == smax) & (ti < scand))
        o_ref[...] = jnp.where(tb, ti, scand)

    return pl.pallas_call(
        body,
        out_shape=jax.ShapeDtypeStruct((1, ROWS), jnp.int32),
    )(sc_val, sc_idx, tc_val, tc_idx)


def kernel(m_logits):
    xt = m_logits.T
    sc_val, sc_idx = _sc_scan(xt)
    tc_val, tc_idx = _tc_scan(xt)
    return _merge(sc_val, sc_idx, tc_val, tc_idx).reshape(ROWS, 1)


# speedup vs baseline: 1.0070x; 1.0070x over previous
import dataclasses
import functools

import jax
import jax.numpy as jnp
from jax import lax
from jax.experimental import pallas as pl
from jax.experimental.pallas import tpu as pltpu
from jax.experimental.pallas import tpu_sc as plsc

ROWS = 128
VOCAB = 100000
LANES = 16
NWORK = 32
CHC = 200
V_SC = 36000
NCHUNK = V_SC // CHC
NJOB = -(-NCHUNK // NWORK)
NBUF = 4
TCB = 2000
NSTREAM = 4
TC_STEPS = (VOCAB - V_SC) // (TCB * NSTREAM)
_INT_MAX = 2**31 - 1


def _sc_scan(xt):
    mesh = plsc.VectorSubcoreMesh(core_axis_name="c", subcore_axis_name="s")

    scratch = [pltpu.VMEM((CHC, ROWS), jnp.float32) for _ in range(NBUF)]
    scratch.append(pltpu.VMEM((ROWS,), jnp.float32))
    scratch.append(pltpu.VMEM((ROWS,), jnp.int32))
    scratch.extend(pltpu.SemaphoreType.DMA for _ in range(NBUF))
    n_full = NCHUNK - (NJOB - 1) * NWORK

    cp = pltpu.CompilerParams()
    if "needs_layout_passes" in pltpu.CompilerParams.__dataclass_fields__:
        cp = dataclasses.replace(cp, needs_layout_passes=False)

    @functools.partial(
        pl.kernel,
        out_type=(
            jax.ShapeDtypeStruct((NWORK, ROWS), jnp.float32),
            jax.ShapeDtypeStruct((NWORK, ROWS), jnp.int32),
        ),
        mesh=mesh,
        scratch_types=scratch,
        compiler_params=cp,
    )
    def sc_argmax(x_hbm, val_hbm, idx_hbm, *rest):
        bufs = rest[:NBUF]
        outv_f = rest[NBUF]
        outv_i = rest[NBUF + 1]
        sems = rest[NBUF + 2:]

        w = lax.axis_index("c") * 16 + lax.axis_index("s")
        has_last = w < n_full

        def _descr(j):
            off = pl.multiple_of((w + j * NWORK) * CHC, 8)
            return pltpu.make_async_copy(
                x_hbm.at[pl.ds(off, CHC), :],
                bufs[j % NBUF],
                sems[j % NBUF],
            )

        def issue(j):
            copy = _descr(j)
            copy.start()
            return copy

        def guarded_issue(j):
            if j < NJOB - 1:
                return issue(j)

            @pl.when(has_last)
            def _():
                issue(j)

        copies = {}
        for j in range(min(NBUF - 1, NJOB)):
            copies[j] = guarded_issue(j)

        neg_inf = jnp.full((LANES,), -jnp.inf, dtype=jnp.float32)
        zero_i = jnp.zeros((LANES,), dtype=jnp.int32)

        rms = [neg_inf] * 8
        ris = [zero_i] * 8

        for j in range(NJOB):
            nxt = j + (NBUF - 1)
            if nxt < NJOB:
                copies[nxt] = guarded_issue(nxt)
            last = j == NJOB - 1
            if not last:
                copies[j].wait()
            else:

                @pl.when(has_last)
                def _(j=j):
                    _descr(j).wait()

            buf = bufs[j % NBUF]
            cbase = (w + j * NWORK) * CHC
            madd = None
            if last:
                madd = jnp.where(
                    has_last, jnp.float32(0), jnp.float32(-jnp.inf)
                )

            def body(i, carry, buf=buf, cbase=cbase, madd=madd):
                c_rms, c_ris = carry
                c_rms, c_ris = list(c_rms), list(c_ris)
                col = jnp.broadcast_to(cbase + i, (LANES,)).astype(jnp.int32)
                for k in range(8):
                    v = buf[i, pl.ds(k * LANES, LANES)]
                    if madd is not None:
                        v = v + madd
                    m = v > c_rms[k]
                    c_rms[k] = jnp.where(m, v, c_rms[k])
                    c_ris[k] = jnp.where(m, col, c_ris[k])
                return tuple(c_rms), tuple(c_ris)

            rms_t, ris_t = lax.fori_loop(0, CHC, body, (tuple(rms), tuple(ris)))
            rms, ris = list(rms_t), list(ris_t)

        for k in range(8):
            outv_f[pl.ds(k * LANES, LANES)] = rms[k]
            outv_i[pl.ds(k * LANES, LANES)] = ris[k]
        pltpu.sync_copy(outv_f, val_hbm.at[w])
        pltpu.sync_copy(outv_i, idx_hbm.at[w])

    return sc_argmax(xt)


def _tc_scan(xt):
    def body(*refs):
        x_refs = refs[:NSTREAM]
        val_ref, idx_ref = refs[NSTREAM:NSTREAM + 2]
        accs = refs[NSTREAM + 2:]
        av = accs[:NSTREAM]
        ai = accs[NSTREAM:]
        pid = pl.program_id(0)

        @pl.when(pid == 0)
        def _():
            for a in range(NSTREAM):
                av[a][...] = jnp.full((8, ROWS), -jnp.inf, dtype=jnp.float32)
                ai[a][...] = jnp.zeros((8, ROWS), dtype=jnp.int32)

        sub_iota = lax.broadcasted_iota(jnp.int32, (8, ROWS), 0)
        cv = [av[a][...] for a in range(NSTREAM)]
        ci = [ai[a][...] for a in range(NSTREAM)]
        for s in range(NSTREAM):
            x = x_refs[s][...]
            base = V_SC + (pid * NSTREAM + s) * TCB
            for j in range(TCB // 8):
                xv = x[j * 8:(j + 1) * 8, :]
                iv = sub_iota + (base + j * 8)
                m = xv > cv[s]
                cv[s] = jnp.where(m, xv, cv[s])
                ci[s] = jnp.where(m, iv, ci[s])
        for a in range(NSTREAM):
            av[a][...] = cv[a]
            ai[a][...] = ci[a]

        @pl.when(pid == TC_STEPS - 1)
        def _():
            rv, ri = cv[0], ci[0]
            for a in range(1, NSTREAM):
                tb = (cv[a] > rv) | ((cv[a] == rv) & (ci[a] < ri))
                rv = jnp.where(tb, cv[a], rv)
                ri = jnp.where(tb, ci[a], ri)
            row_max = jnp.max(rv, axis=0, keepdims=True)
            cand = jnp.where(rv == row_max, ri, _INT_MAX)
            val_ref[...] = row_max
            idx_ref[...] = jnp.min(cand, axis=0, keepdims=True)

    in_specs = [
        pl.BlockSpec(
            (TCB, ROWS),
            functools.partial(
                lambda i, s: (V_SC // TCB + i * NSTREAM + s, 0), s=s
            ),
        )
        for s in range(NSTREAM)
    ]
    return pl.pallas_call(
        body,
        grid=(TC_STEPS,),
        out_shape=(
            jax.ShapeDtypeStruct((1, ROWS), jnp.float32),
            jax.ShapeDtypeStruct((1, ROWS), jnp.int32),
        ),
        in_specs=in_specs,
        out_specs=(
            pl.BlockSpec((1, ROWS), lambda i: (0, 0)),
            pl.BlockSpec((1, ROWS), lambda i: (0, 0)),
        ),
        scratch_shapes=[pltpu.VMEM((8, ROWS), jnp.float32)] * NSTREAM
        + [pltpu.VMEM((8, ROWS), jnp.int32)] * NSTREAM,
    )(*([xt] * NSTREAM))


def _merge(sc_val, sc_idx, tc_val, tc_idx):
    def body(sv_ref, si_ref, tv_ref, ti_ref, o_ref):
        sv = sv_ref[...]
        si = si_ref[...]
        smax = jnp.max(sv, axis=0, keepdims=True)
        scand = jnp.min(
            jnp.where(sv == smax, si, _INT_MAX), axis=0, keepdims=True
        )
        tv = tv_ref[...]
        ti = ti_ref[...]
        tb = (tv > smax) | ((tv == smax) & (ti < scand))
        o_ref[...] = jnp.where(tb, ti, scand)

    return pl.pallas_call(
        body,
        out_shape=jax.ShapeDtypeStruct((1, ROWS), jnp.int32),
    )(sc_val, sc_idx, tc_val, tc_idx)


def kernel(m_logits):
    xt = m_logits.T
    sc_val, sc_idx = _sc_scan(xt)
    tc_val, tc_idx = _tc_scan(xt)
    return _merge(sc_val, sc_idx, tc_val, tc_idx).reshape(ROWS, 1)
